# baseline (speedup 1.0000x reference)
import jax
import jax.numpy as jnp
from jax import lax
from jax.experimental import pallas as pl
from jax.experimental.pallas import tpu as pltpu
from jax.experimental.pallas import tpu_sc as plsc

BATCH = 16384
HIST = 200
EMBED_DIM = 64

NUM_CORES = 2
NUM_SUBCORES = 16
NUM_WORKERS = NUM_CORES * NUM_SUBCORES

TOTAL = BATCH * HIST
HALF = TOTAL // 2

CHUNK = 256
DPW = HALF // NUM_WORKERS
SUPER = NUM_SUBCORES * CHUNK
SPC = HALF // NUM_CORES
STEPS = DPW // CHUNK


def _sc_body(actions_hbm, table_hbm, out_hbm, table_sh, big_sh,
             idx_d, rows_d, idx_s, rows_s,
             sem_gd, sem_od, sem_id, sem_gs, sem_os, sem_is):
    cid = lax.axis_index("c")
    sid = lax.axis_index("s")
    wid = sid * NUM_CORES + cid
    dbase0 = wid * DPW
    sbase0 = HALF + cid * SPC

    @pl.when(sid == 0)
    def _():
        pltpu.sync_copy(table_hbm, table_sh)
    plsc.subcore_barrier()

    for b in range(2):
        pltpu.sync_copy(
            actions_hbm.at[pl.ds(dbase0 + b * CHUNK, CHUNK)], idx_d[b])
        pltpu.sync_copy(
            actions_hbm.at[pl.ds(sbase0 + b * SUPER + sid * CHUNK, CHUNK)],
            idx_s[b])

    @pl.loop(0, STEPS // 2)
    def _pair(p):
        for b in range(2):
            t = 2 * p + b
            dbase = dbase0 + t * CHUNK
            sbase = sbase0 + t * SUPER
            stile = sbase + sid * CHUNK

            @pl.when(p > 0)
            def _():
                pltpu.make_async_copy(
                    actions_hbm.at[pl.ds(dbase, CHUNK)], idx_d[b],
                    sem_id[b]).wait()
                pltpu.make_async_copy(
                    rows_d[b], out_hbm.at[pl.ds(dbase, CHUNK)],
                    sem_od[b]).wait()
            pltpu.async_copy(table_sh.at[idx_d[b]], rows_d[b],
                             sem_gd[b]).wait()
            pltpu.make_async_copy(
                rows_d[b], out_hbm.at[pl.ds(dbase, CHUNK)],
                sem_od[b]).start()

            @pl.when(p > 0)
            def _():
                pltpu.make_async_copy(
                    actions_hbm.at[pl.ds(stile, CHUNK)], idx_s[b],
                    sem_is[b]).wait()
            pltpu.async_copy(table_sh.at[idx_s[b]], rows_s[b],
                             sem_gs[b]).wait()

            @pl.when(jnp.logical_and(p > 0, sid == 0))
            def _():
                pltpu.make_async_copy(
                    big_sh.at[b], out_hbm.at[pl.ds(sbase, SUPER)],
                    sem_os[b]).wait()
            plsc.subcore_barrier()
            pltpu.sync_copy(rows_s[b],
                            big_sh.at[b].at[pl.ds(sid * CHUNK, CHUNK)])
            plsc.subcore_barrier()

            @pl.when(sid == 0)
            def _():
                pltpu.make_async_copy(
                    big_sh.at[b], out_hbm.at[pl.ds(sbase, SUPER)],
                    sem_os[b]).start()

            @pl.when(p < STEPS // 2 - 1)
            def _():
                pltpu.make_async_copy(
                    actions_hbm.at[pl.ds(dbase + 2 * CHUNK, CHUNK)],
                    idx_d[b], sem_id[b]).start()
                pltpu.make_async_copy(
                    actions_hbm.at[pl.ds(stile + 2 * SUPER, CHUNK)],
                    idx_s[b], sem_is[b]).start()

    for b in range(2):
        t = STEPS - 2 + b
        pltpu.make_async_copy(
            rows_d[b], out_hbm.at[pl.ds(dbase0 + t * CHUNK, CHUNK)],
            sem_od[b]).wait()

    @pl.when(sid == 0)
    def _():
        for b in range(2):
            t = STEPS - 2 + b
            pltpu.make_async_copy(
                big_sh.at[b], out_hbm.at[pl.ds(sbase0 + t * SUPER, SUPER)],
                sem_os[b]).wait()
    plsc.subcore_barrier()


@jax.jit
def _embed_sc(actions_flat, table8):
    mesh = plsc.VectorSubcoreMesh(core_axis_name="c", subcore_axis_name="s")

    def body(actions_hbm, table_hbm, out_hbm, table_sh, big_sh, *rest):
        groups = [rest[i * 2:(i + 1) * 2] for i in range(10)]
        _sc_body(actions_hbm, table_hbm, out_hbm, table_sh, big_sh,
                 *groups)

    scratch = [
        pltpu.VMEM_SHARED((8, EMBED_DIM), jnp.float32),
        pltpu.VMEM_SHARED((2, SUPER, EMBED_DIM), jnp.float32),
    ]
    scratch += [pltpu.VMEM((CHUNK,), jnp.int32) for _ in range(2)]
    scratch += [pltpu.VMEM((CHUNK, EMBED_DIM), jnp.float32)
                for _ in range(2)]
    scratch += [pltpu.VMEM((CHUNK,), jnp.int32) for _ in range(2)]
    scratch += [pltpu.VMEM((CHUNK, EMBED_DIM), jnp.float32)
                for _ in range(2)]
    scratch += [pltpu.SemaphoreType.DMA for _ in range(12)]

    return pl.kernel(
        body,
        out_type=jax.ShapeDtypeStruct((TOTAL, EMBED_DIM), jnp.float32),
        mesh=mesh,
        scratch_types=scratch,
        compiler_params=pltpu.CompilerParams(use_tc_tiling_on_sc=False),
    )(actions_flat, table8)


def kernel(actions, embedding_weight):
    table8 = jnp.concatenate(
        [jnp.zeros((1, EMBED_DIM), jnp.float32), embedding_weight], axis=0)
    out = _embed_sc(actions.reshape(TOTAL), table8)
    return out.reshape(BATCH, HIST, EMBED_DIM)


# speedup vs baseline: 1.0754x; 1.0754x over previous
import jax
import jax.numpy as jnp
from jax import lax
from jax.experimental import pallas as pl
from jax.experimental.pallas import tpu as pltpu
from jax.experimental.pallas import tpu_sc as plsc

BATCH = 16384
HIST = 200
EMBED_DIM = 64

NUM_CORES = 2
NUM_SUBCORES = 16
NUM_WORKERS = NUM_CORES * NUM_SUBCORES

TOTAL = BATCH * HIST
ROWS_PER_WORKER = TOTAL // NUM_WORKERS

NBUF = 2
CHUNK = 512
CHUNKS = ROWS_PER_WORKER // CHUNK
NROUND = CHUNKS // NBUF


def _sc_body(actions_hbm, table_hbm, out_hbm, table_sh,
             idx_v, rows_v, sem_g, sem_o, sem_i):
    cid = lax.axis_index("c")
    sid = lax.axis_index("s")
    wid = sid * NUM_CORES + cid
    wbase = wid * ROWS_PER_WORKER

    @pl.when(sid == 0)
    def _():
        pltpu.sync_copy(table_hbm, table_sh)
    plsc.subcore_barrier()

    for b in range(NBUF):
        pltpu.sync_copy(actions_hbm.at[pl.ds(wbase + b * CHUNK, CHUNK)],
                        idx_v[b])

    @pl.loop(0, NROUND)
    def _round(t):
        for b in range(NBUF):
            c = t * NBUF + b
            base = wbase + c * CHUNK

            @pl.when(t > 0)
            def _():
                pltpu.make_async_copy(
                    actions_hbm.at[pl.ds(base, CHUNK)], idx_v[b],
                    sem_i[b]).wait()
                pltpu.make_async_copy(
                    rows_v[b], out_hbm.at[pl.ds(base, CHUNK)],
                    sem_o[b]).wait()

            pltpu.async_copy(table_sh.at[idx_v[b]], rows_v[b],
                             sem_g[b]).wait()
            pltpu.make_async_copy(
                rows_v[b], out_hbm.at[pl.ds(base, CHUNK)], sem_o[b]).start()

            @pl.when(t < NROUND - 1)
            def _():
                pltpu.make_async_copy(
                    actions_hbm.at[pl.ds(base + NBUF * CHUNK, CHUNK)],
                    idx_v[b], sem_i[b]).start()

    for b in range(NBUF):
        c = CHUNKS - NBUF + b
        pltpu.make_async_copy(
            rows_v[b], out_hbm.at[pl.ds(wbase + c * CHUNK, CHUNK)],
            sem_o[b]).wait()


@jax.jit
def _embed_sc(actions_flat, table8):
    mesh = plsc.VectorSubcoreMesh(core_axis_name="c", subcore_axis_name="s")

    def body(actions_hbm, table_hbm, out_hbm, table_sh, *rest):
        idx_v = rest[0:NBUF]
        rows_v = rest[NBUF:2 * NBUF]
        sem_g = rest[2 * NBUF:3 * NBUF]
        sem_o = rest[3 * NBUF:4 * NBUF]
        sem_i = rest[4 * NBUF:5 * NBUF]
        _sc_body(actions_hbm, table_hbm, out_hbm, table_sh,
                 idx_v, rows_v, sem_g, sem_o, sem_i)

    scratch = [pltpu.VMEM_SHARED((8, EMBED_DIM), jnp.float32)]
    scratch += [pltpu.VMEM((CHUNK,), jnp.int32) for _ in range(NBUF)]
    scratch += [pltpu.VMEM((CHUNK, EMBED_DIM), jnp.float32)
                for _ in range(NBUF)]
    scratch += [pltpu.SemaphoreType.DMA for _ in range(3 * NBUF)]

    return pl.kernel(
        body,
        out_type=jax.ShapeDtypeStruct((TOTAL, EMBED_DIM), jnp.float32),
        mesh=mesh,
        scratch_types=scratch,
        compiler_params=pltpu.CompilerParams(use_tc_tiling_on_sc=False),
    )(actions_flat, table8)


def kernel(actions, embedding_weight):
    table8 = jnp.concatenate(
        [jnp.zeros((1, EMBED_DIM), jnp.float32), embedding_weight], axis=0)
    out = _embed_sc(actions.reshape(TOTAL), table8)
    return out.reshape(BATCH, HIST, EMBED_DIM)
